# baseline (speedup 1.0000x reference)
import functools

import jax
import jax.numpy as jnp
from jax import lax
from jax.experimental import pallas as pl
from jax.experimental.pallas import tpu as pltpu
from jax.experimental.pallas import tpu_sc as plsc

_NUM_CORES = 2
_NUM_SUBCORES = 16
_NUM_WORKERS = _NUM_CORES * _NUM_SUBCORES


@functools.partial(jax.jit, static_argnums=(1, 2))
def _copy_rows(emb_weight, seq_len, dim):
    rows_per_w = seq_len // _NUM_WORKERS
    mesh = plsc.VectorSubcoreMesh(core_axis_name="c", subcore_axis_name="s")

    n_buf = 3
    chunk = 32
    n_chunks = rows_per_w // chunk
    n_buf_s = 2
    chunk_s = 16
    n_chunks_s = rows_per_w // chunk_s

    @functools.partial(
        pl.kernel,
        mesh=mesh,
        out_type=jax.ShapeDtypeStruct((seq_len, dim), emb_weight.dtype),
        scratch_types=(
            [
                pltpu.VMEM((n_buf, chunk, dim), jnp.float32),
                pltpu.VMEM_SHARED(
                    (_NUM_SUBCORES // 2, n_buf_s, chunk_s, dim), jnp.float32
                ),
            ]
            + [pltpu.SemaphoreType.DMA] * (2 * n_buf)
        ),
    )
    def copy_kernel(emb_hbm, out_hbm, tile_buf, shared_buf, *sems):
        s = lax.axis_index("s")
        wid = s * _NUM_CORES + lax.axis_index("c")
        base = wid * rows_per_w
        rsems = list(sems[:n_buf])
        wsems = list(sems[n_buf:])

        def run_pipeline(buf_at, nb, ck, nc):
            def start_read(j):
                return pltpu.async_copy(
                    emb_hbm.at[pl.ds(base + j * ck, ck)],
                    buf_at(j % nb),
                    rsems[j % nb],
                )

            def start_write(j):
                return pltpu.async_copy(
                    buf_at(j % nb),
                    out_hbm.at[pl.ds(base + j * ck, ck)],
                    wsems[j % nb],
                )

            rh = [start_read(b) for b in range(nb)]
            wh = [None] * nb
            for j in range(nc):
                b = j % nb
                rh[b].wait()
                wh[b] = start_write(j)
                if j + nb < nc:
                    wh[b].wait()
                    rh[b] = start_read(j + nb)
            for b in range(nb):
                wh[b].wait()

        @pl.when(s % 2 == 0)
        def _():
            run_pipeline(lambda b: tile_buf.at[b], n_buf, chunk, n_chunks)

        @pl.when(s % 2 == 1)
        def _():
            run_pipeline(
                lambda b: shared_buf.at[s // 2, b], n_buf_s, chunk_s, n_chunks_s
            )

    return copy_kernel(emb_weight)


def kernel(x, emb_weight):
    seq_len = x.shape[1]
    return _copy_rows(emb_weight, seq_len, emb_weight.shape[1])
